# baseline (speedup 1.0000x reference)
import functools

import numpy as np
import jax
import jax.numpy as jnp
from jax import lax
from jax.experimental import pallas as pl
from jax.experimental.pallas import tpu as pltpu
from jax.experimental.pallas import tpu_sc as plsc

_BS = 128
_N = 256
_L = 16
_NC = 2
_NS = 16
_SAMPLES_PER_SUBCORE = _BS // (_NC * _NS)
_T_DOUBLE = 9

_COEFF = 0.01
_LAM_PATH = np.float32(2.0 - 2.0 * np.cos(np.pi / _N))
_LAM_CYCLE = np.float32(2.0 - 2.0 * np.cos(2.0 * np.pi / _N))
_OPTI = _LAM_CYCLE


_B_BLK = 16


def _tc_body(x_ref, t_ref, bce_ref, i1_ref, i2_ref):
    x = x_ref[...]
    t = t_ref[...]
    ax = jnp.abs(x)
    l = jnp.log(1.0 + jnp.exp(-ax))
    px = jnp.minimum(jnp.maximum(x, 0.0) + l, 100.0)
    xc = jnp.clip(x, -100.0, 100.0)
    s = (jnp.sum(px, axis=(1, 2), keepdims=True)
         - jnp.sum(t * xc, axis=(1, 2), keepdims=True)) * np.float32(1.0 / (_N * _N))
    bce_ref[...] = jnp.broadcast_to(s, (_B_BLK, 1, _L))

    iota = lax.broadcasted_iota(jnp.int32, (_B_BLK, _N, _N), 2)
    m1 = jnp.max(x, axis=2, keepdims=True)
    i1 = jnp.min(jnp.where(x == m1, iota, _N), axis=2, keepdims=True)
    x2 = jnp.where(iota == i1, -jnp.inf, x)
    m2 = jnp.max(x2, axis=2, keepdims=True)
    i2 = jnp.min(jnp.where(x2 == m2, iota, _N), axis=2, keepdims=True)
    i1_ref[...] = i1
    i2_ref[...] = i2


def _tc_call(raw_scores, target):
    return pl.pallas_call(
        _tc_body,
        grid=(_BS // _B_BLK,),
        in_specs=[
            pl.BlockSpec((_B_BLK, _N, _N), lambda b: (b, 0, 0)),
            pl.BlockSpec((_B_BLK, _N, _N), lambda b: (b, 0, 0)),
        ],
        out_specs=[
            pl.BlockSpec((_B_BLK, 1, _L), lambda b: (b, 0, 0)),
            pl.BlockSpec((_B_BLK, _N, 1), lambda b: (b, 0, 0)),
            pl.BlockSpec((_B_BLK, _N, 1), lambda b: (b, 0, 0)),
        ],
        out_shape=[
            jax.ShapeDtypeStruct((_BS, 1, _L), jnp.float32),
            jax.ShapeDtypeStruct((_BS, _N, 1), jnp.int32),
            jax.ShapeDtypeStruct((_BS, _N, 1), jnp.int32),
        ],
        compiler_params=pltpu.CompilerParams(
            dimension_semantics=("parallel",),
        ),
    )(raw_scores, target)


def _sc_body(i1_hbm, i2_hbm, bce_hbm, out_hbm,
             a_v, b_v, n1_v, n2_v, mval_v, nxt_v, bce_v, res_v, all_v, fv_v,
             part_sh):
    cid = lax.axis_index("c")
    sid = lax.axis_index("s")
    lanes = lax.iota(jnp.int32, _L)
    n_chunks = _N // _L

    partial = jnp.float32(0.0)
    for k in range(_SAMPLES_PER_SUBCORE):
        b = (cid * _NS + sid) * _SAMPLES_PER_SUBCORE + k
        pltpu.sync_copy(i1_hbm.at[b], a_v)
        pltpu.sync_copy(i2_hbm.at[b], b_v)
        pltpu.sync_copy(bce_hbm.at[b, 0], bce_v)

        def setup_chunk(i, deg):
            base = i * _L
            iv = base + lanes
            c1 = a_v[pl.ds(base, _L)]
            c2 = b_v[pl.ds(base, _L)]
            a_c1 = plsc.load_gather(a_v, [c1])
            b_c1 = plsc.load_gather(b_v, [c1])
            a_c2 = plsc.load_gather(a_v, [c2])
            b_c2 = plsc.load_gather(b_v, [c2])
            r1 = ((a_c1 == iv) | (b_c1 == iv)) & (c1 != iv)
            r2 = ((a_c2 == iv) | (b_c2 == iv)) & (c2 != iv)
            n1_v[pl.ds(base, _L)] = jnp.where(r1, c1, iv)
            n2_v[pl.ds(base, _L)] = jnp.where(r2, c2, iv)
            return deg + r1.astype(jnp.int32) + r2.astype(jnp.int32)

        deg = lax.fori_loop(0, n_chunks, setup_chunk,
                            jnp.zeros((_L,), jnp.int32))
        deg_total = jnp.sum(deg)

        fv_v[...] = jnp.zeros((_L,), jnp.float32)

        @pl.when(deg_total >= 2 * (_N - 1))
        def _component_search():
            _run_component_search(deg_total, lanes, n_chunks,
                                  n1_v, n2_v, mval_v, nxt_v, fv_v)

        fv = fv_v[...][0]
        d = fv - _OPTI
        partial = partial + bce_v[...][0] + jnp.float32(_COEFF) * d * d

    res_v[...] = jnp.where(lanes == 0, partial, jnp.float32(0.0))
    pltpu.sync_copy(res_v, part_sh.at[pl.ds(sid * _L, _L)])
    plsc.subcore_barrier()

    @pl.when(sid == 0)
    def _():
        pltpu.sync_copy(part_sh, all_v)
        acc = jnp.zeros((_L,), jnp.float32)
        for j in range(_NS):
            acc = acc + all_v[pl.ds(j * _L, _L)]
        res_v[...] = acc * jnp.float32(1.0 / _BS)
        pltpu.sync_copy(res_v, out_hbm.at[cid])


def _run_component_search(deg_total, lanes, n_chunks,
                          n1_v, n2_v, mval_v, nxt_v, fv_v):
        def edge_chunk(i, _):
            base = i * _L
            iv = base + lanes
            n1 = n1_v[pl.ds(base, _L)]
            n2 = n2_v[pl.ds(base, _L)]
            n1_of_n1 = plsc.load_gather(n1_v, [n1])
            n1_of_n2 = plsc.load_gather(n1_v, [n2])
            nxt1 = jnp.where(n1 == iv, iv,
                             jnp.where(n1_of_n1 == iv, n1 + _N, n1))
            nxt2 = jnp.where(n2 == iv, iv + _N,
                             jnp.where(n1_of_n2 == iv, n2 + _N, n2))
            mval_v[pl.ds(base, _L)] = n1
            mval_v[pl.ds(base + _N, _L)] = n2
            nxt_v[pl.ds(base, _L)] = nxt1
            nxt_v[pl.ds(base + _N, _L)] = nxt2
            return 0

        lax.fori_loop(0, n_chunks, edge_chunk, 0)

        def double_round(t, _):
            def dbl_chunk(i, _):
                base = i * _L
                cm = mval_v[pl.ds(base, _L)]
                nx = nxt_v[pl.ds(base, _L)]
                gm = plsc.load_gather(mval_v, [nx])
                gn = plsc.load_gather(nxt_v, [nx])
                mval_v[pl.ds(base, _L)] = jnp.minimum(cm, gm)
                nxt_v[pl.ds(base, _L)] = gn
                return 0
            lax.fori_loop(0, 2 * _N // _L, dbl_chunk, 0)
            return 0

        lax.fori_loop(0, _T_DOUBLE, double_round, 0)

        def final_chunk(i, mx):
            base = i * _L
            iv = base + lanes
            cm = jnp.minimum(iv, jnp.minimum(mval_v[pl.ds(base, _L)],
                                             mval_v[pl.ds(base + _N, _L)]))
            return jnp.maximum(mx, cm)

        mx = lax.fori_loop(0, n_chunks, final_chunk,
                           jnp.zeros((_L,), jnp.int32))
        connected = jnp.max(mx) == 0
        fv = jnp.where(connected,
                       jnp.where(deg_total == 2 * _N, _LAM_CYCLE, _LAM_PATH),
                       jnp.float32(0.0))
        fv_v[...] = jnp.where(lanes == 0, fv, jnp.float32(0.0))


@functools.cache
def _get_sc_call():
    return pl.kernel(
        _sc_body,
        out_type=jax.ShapeDtypeStruct((_NC, _L), jnp.float32),
        mesh=plsc.VectorSubcoreMesh(core_axis_name="c", subcore_axis_name="s",
                                    num_cores=_NC, num_subcores=_NS),
        scratch_types=[
            pltpu.VMEM((_N,), jnp.int32),
            pltpu.VMEM((_N,), jnp.int32),
            pltpu.VMEM((_N,), jnp.int32),
            pltpu.VMEM((_N,), jnp.int32),
            pltpu.VMEM((2 * _N,), jnp.int32),
            pltpu.VMEM((2 * _N,), jnp.int32),
            pltpu.VMEM((_L,), jnp.float32),
            pltpu.VMEM((_L,), jnp.float32),
            pltpu.VMEM((_NS * _L,), jnp.float32),
            pltpu.VMEM((_L,), jnp.float32),
            pltpu.VMEM_SHARED((_NS * _L,), jnp.float32),
        ],
        compiler_params=pltpu.CompilerParams(needs_layout_passes=False),
    )


@jax.jit
def kernel(raw_scores, target):
    bce, i1, i2 = _tc_call(raw_scores, target)
    out = _get_sc_call()(i1.reshape(_BS, _N), i2.reshape(_BS, _N), bce)
    return out[0, 0] + out[1, 0]


# speedup vs baseline: 1.0168x; 1.0168x over previous
import functools

import numpy as np
import jax
import jax.numpy as jnp
from jax import lax
from jax.experimental import pallas as pl
from jax.experimental.pallas import tpu as pltpu
from jax.experimental.pallas import tpu_sc as plsc

_BS = 128
_N = 256
_L = 16
_NC = 2
_NS = 16
_SAMPLES_PER_SUBCORE = _BS // (_NC * _NS)
_T_DOUBLE = 9

_COEFF = 0.01
_LAM_PATH = np.float32(2.0 - 2.0 * np.cos(np.pi / _N))
_LAM_CYCLE = np.float32(2.0 - 2.0 * np.cos(2.0 * np.pi / _N))
_OPTI = _LAM_CYCLE


_B_BLK = 8


def _tc_body(x_ref, t_ref, bce_ref, i1_ref, i2_ref):
    x = x_ref[...]
    t = t_ref[...]
    ax = jnp.abs(x)
    l = jnp.log(1.0 + jnp.exp(-ax))
    px = jnp.minimum(jnp.maximum(x, 0.0) + l, 100.0)
    xc = jnp.clip(x, -100.0, 100.0)
    s = (jnp.sum(px, axis=(1, 2), keepdims=True)
         - jnp.sum(t * xc, axis=(1, 2), keepdims=True)) * np.float32(1.0 / (_N * _N))
    bce_ref[...] = jnp.broadcast_to(s, (_B_BLK, 1, _L))

    iota = lax.broadcasted_iota(jnp.int32, (_B_BLK, _N, _N), 2)
    m1 = jnp.max(x, axis=2, keepdims=True)
    i1 = jnp.min(jnp.where(x == m1, iota, _N), axis=2, keepdims=True)
    x2 = jnp.where(iota == i1, -jnp.inf, x)
    m2 = jnp.max(x2, axis=2, keepdims=True)
    i2 = jnp.min(jnp.where(x2 == m2, iota, _N), axis=2, keepdims=True)
    i1_ref[...] = i1
    i2_ref[...] = i2


def _tc_call(raw_scores, target):
    return pl.pallas_call(
        _tc_body,
        grid=(_BS // _B_BLK,),
        in_specs=[
            pl.BlockSpec((_B_BLK, _N, _N), lambda b: (b, 0, 0)),
            pl.BlockSpec((_B_BLK, _N, _N), lambda b: (b, 0, 0)),
        ],
        out_specs=[
            pl.BlockSpec((_B_BLK, 1, _L), lambda b: (b, 0, 0)),
            pl.BlockSpec((_B_BLK, _N, 1), lambda b: (b, 0, 0)),
            pl.BlockSpec((_B_BLK, _N, 1), lambda b: (b, 0, 0)),
        ],
        out_shape=[
            jax.ShapeDtypeStruct((_BS, 1, _L), jnp.float32),
            jax.ShapeDtypeStruct((_BS, _N, 1), jnp.int32),
            jax.ShapeDtypeStruct((_BS, _N, 1), jnp.int32),
        ],
        compiler_params=pltpu.CompilerParams(
            dimension_semantics=("parallel",),
        ),
    )(raw_scores, target)


def _sc_body(i1_hbm, i2_hbm, bce_hbm, out_hbm,
             a_v, b_v, n1_v, n2_v, mval_v, nxt_v, bce_v, res_v, all_v, fv_v,
             part_sh):
    cid = lax.axis_index("c")
    sid = lax.axis_index("s")
    lanes = lax.iota(jnp.int32, _L)
    n_chunks = _N // _L

    partial = jnp.float32(0.0)
    for k in range(_SAMPLES_PER_SUBCORE):
        b = (cid * _NS + sid) * _SAMPLES_PER_SUBCORE + k
        pltpu.sync_copy(i1_hbm.at[b], a_v)
        pltpu.sync_copy(i2_hbm.at[b], b_v)
        pltpu.sync_copy(bce_hbm.at[b, 0], bce_v)

        def setup_chunk(i, deg):
            base = i * _L
            iv = base + lanes
            c1 = a_v[pl.ds(base, _L)]
            c2 = b_v[pl.ds(base, _L)]
            a_c1 = plsc.load_gather(a_v, [c1])
            b_c1 = plsc.load_gather(b_v, [c1])
            a_c2 = plsc.load_gather(a_v, [c2])
            b_c2 = plsc.load_gather(b_v, [c2])
            r1 = ((a_c1 == iv) | (b_c1 == iv)) & (c1 != iv)
            r2 = ((a_c2 == iv) | (b_c2 == iv)) & (c2 != iv)
            n1_v[pl.ds(base, _L)] = jnp.where(r1, c1, iv)
            n2_v[pl.ds(base, _L)] = jnp.where(r2, c2, iv)
            return deg + r1.astype(jnp.int32) + r2.astype(jnp.int32)

        deg = lax.fori_loop(0, n_chunks, setup_chunk,
                            jnp.zeros((_L,), jnp.int32))
        deg_total = jnp.sum(deg)

        fv_v[...] = jnp.zeros((_L,), jnp.float32)

        @pl.when(deg_total >= 2 * (_N - 1))
        def _component_search():
            _run_component_search(deg_total, lanes, n_chunks,
                                  n1_v, n2_v, mval_v, nxt_v, fv_v)

        fv = fv_v[...][0]
        d = fv - _OPTI
        partial = partial + bce_v[...][0] + jnp.float32(_COEFF) * d * d

    res_v[...] = jnp.where(lanes == 0, partial, jnp.float32(0.0))
    pltpu.sync_copy(res_v, part_sh.at[pl.ds(sid * _L, _L)])
    plsc.subcore_barrier()

    @pl.when(sid == 0)
    def _():
        pltpu.sync_copy(part_sh, all_v)
        acc = jnp.zeros((_L,), jnp.float32)
        for j in range(_NS):
            acc = acc + all_v[pl.ds(j * _L, _L)]
        res_v[...] = acc * jnp.float32(1.0 / _BS)
        pltpu.sync_copy(res_v, out_hbm.at[cid])


def _run_component_search(deg_total, lanes, n_chunks,
                          n1_v, n2_v, mval_v, nxt_v, fv_v):
        def edge_chunk(i, _):
            base = i * _L
            iv = base + lanes
            n1 = n1_v[pl.ds(base, _L)]
            n2 = n2_v[pl.ds(base, _L)]
            n1_of_n1 = plsc.load_gather(n1_v, [n1])
            n1_of_n2 = plsc.load_gather(n1_v, [n2])
            nxt1 = jnp.where(n1 == iv, iv,
                             jnp.where(n1_of_n1 == iv, n1 + _N, n1))
            nxt2 = jnp.where(n2 == iv, iv + _N,
                             jnp.where(n1_of_n2 == iv, n2 + _N, n2))
            mval_v[pl.ds(base, _L)] = n1
            mval_v[pl.ds(base + _N, _L)] = n2
            nxt_v[pl.ds(base, _L)] = nxt1
            nxt_v[pl.ds(base + _N, _L)] = nxt2
            return 0

        lax.fori_loop(0, n_chunks, edge_chunk, 0)

        def double_round(t, _):
            def dbl_chunk(i, _):
                base = i * _L
                cm = mval_v[pl.ds(base, _L)]
                nx = nxt_v[pl.ds(base, _L)]
                gm = plsc.load_gather(mval_v, [nx])
                gn = plsc.load_gather(nxt_v, [nx])
                mval_v[pl.ds(base, _L)] = jnp.minimum(cm, gm)
                nxt_v[pl.ds(base, _L)] = gn
                return 0
            lax.fori_loop(0, 2 * _N // _L, dbl_chunk, 0)
            return 0

        lax.fori_loop(0, _T_DOUBLE, double_round, 0)

        def final_chunk(i, mx):
            base = i * _L
            iv = base + lanes
            cm = jnp.minimum(iv, jnp.minimum(mval_v[pl.ds(base, _L)],
                                             mval_v[pl.ds(base + _N, _L)]))
            return jnp.maximum(mx, cm)

        mx = lax.fori_loop(0, n_chunks, final_chunk,
                           jnp.zeros((_L,), jnp.int32))
        connected = jnp.max(mx) == 0
        fv = jnp.where(connected,
                       jnp.where(deg_total == 2 * _N, _LAM_CYCLE, _LAM_PATH),
                       jnp.float32(0.0))
        fv_v[...] = jnp.where(lanes == 0, fv, jnp.float32(0.0))


@functools.cache
def _get_sc_call():
    return pl.kernel(
        _sc_body,
        out_type=jax.ShapeDtypeStruct((_NC, _L), jnp.float32),
        mesh=plsc.VectorSubcoreMesh(core_axis_name="c", subcore_axis_name="s",
                                    num_cores=_NC, num_subcores=_NS),
        scratch_types=[
            pltpu.VMEM((_N,), jnp.int32),
            pltpu.VMEM((_N,), jnp.int32),
            pltpu.VMEM((_N,), jnp.int32),
            pltpu.VMEM((_N,), jnp.int32),
            pltpu.VMEM((2 * _N,), jnp.int32),
            pltpu.VMEM((2 * _N,), jnp.int32),
            pltpu.VMEM((_L,), jnp.float32),
            pltpu.VMEM((_L,), jnp.float32),
            pltpu.VMEM((_NS * _L,), jnp.float32),
            pltpu.VMEM((_L,), jnp.float32),
            pltpu.VMEM_SHARED((_NS * _L,), jnp.float32),
        ],
        compiler_params=pltpu.CompilerParams(needs_layout_passes=False),
    )


@jax.jit
def kernel(raw_scores, target):
    bce, i1, i2 = _tc_call(raw_scores, target)
    out = _get_sc_call()(i1.reshape(_BS, _N), i2.reshape(_BS, _N), bce)
    return out[0, 0] + out[1, 0]


# speedup vs baseline: 1.1288x; 1.1102x over previous
import functools

import numpy as np
import jax
import jax.numpy as jnp
from jax import lax
from jax.experimental import pallas as pl
from jax.experimental.pallas import tpu as pltpu
from jax.experimental.pallas import tpu_sc as plsc

_BS = 128
_N = 256
_L = 16
_NC = 2
_NS = 16
_SAMPLES_PER_SUBCORE = _BS // (_NC * _NS)
_T_DOUBLE = 9
_PACK = _N + _L

_COEFF = 0.01
_LAM_PATH = np.float32(2.0 - 2.0 * np.cos(np.pi / _N))
_LAM_CYCLE = np.float32(2.0 - 2.0 * np.cos(2.0 * np.pi / _N))
_OPTI = _LAM_CYCLE


_B_BLK = 8


def _tc_body(x_ref, t_ref, out_ref):
    x = x_ref[...]
    t = t_ref[...]
    ax = jnp.abs(x)
    l = jnp.log(1.0 + jnp.exp(-ax))
    px = jnp.minimum(jnp.maximum(x, 0.0) + l, 100.0)
    xc = jnp.clip(x, -100.0, 100.0)
    s = (jnp.sum(px, axis=(1, 2), keepdims=True)
         - jnp.sum(t * xc, axis=(1, 2), keepdims=True)) * np.float32(1.0 / (_N * _N))

    iota = lax.broadcasted_iota(jnp.int32, (_B_BLK, _N, _N), 2)
    m1 = jnp.max(x, axis=2, keepdims=True)
    i1 = jnp.min(jnp.where(x == m1, iota, _N), axis=2, keepdims=True)
    x2 = jnp.where(iota == i1, -jnp.inf, x)
    m2 = jnp.max(x2, axis=2, keepdims=True)
    i2 = jnp.min(jnp.where(x2 == m2, iota, _N), axis=2, keepdims=True)

    combo = i1 * 256 + i2
    sbits = lax.bitcast_convert_type(s, jnp.int32)
    pad = jnp.zeros((_B_BLK, _PACK - _N - 1, 1), jnp.int32)
    out_ref[...] = jnp.concatenate([combo, sbits, pad], axis=1)


def _tc_call(raw_scores, target):
    return pl.pallas_call(
        _tc_body,
        grid=(_BS // _B_BLK,),
        in_specs=[
            pl.BlockSpec((_B_BLK, _N, _N), lambda b: (b, 0, 0)),
            pl.BlockSpec((_B_BLK, _N, _N), lambda b: (b, 0, 0)),
        ],
        out_specs=pl.BlockSpec((_B_BLK, _PACK, 1), lambda b: (b, 0, 0)),
        out_shape=jax.ShapeDtypeStruct((_BS, _PACK, 1), jnp.int32),
        compiler_params=pltpu.CompilerParams(
            dimension_semantics=("parallel",),
        ),
    )(raw_scores, target)


def _sc_body(packed_hbm, out_hbm,
             row_v, n1_v, n2_v, mval_v, nxt_v, res_v, all_v, fv_v,
             part_sh):
    cid = lax.axis_index("c")
    sid = lax.axis_index("s")
    lanes = lax.iota(jnp.int32, _L)
    n_chunks = _N // _L

    partial = jnp.float32(0.0)
    for k in range(_SAMPLES_PER_SUBCORE):
        b = (cid * _NS + sid) * _SAMPLES_PER_SUBCORE + k
        pltpu.sync_copy(packed_hbm.at[b], row_v)
        bce_val = plsc.bitcast(row_v[pl.ds(_N, _L)], jnp.float32)[0]

        def setup_chunk(i, deg):
            base = i * _L
            iv = base + lanes
            combo = row_v[pl.ds(base, _L)]
            c1 = lax.shift_right_logical(combo, 8)
            c2 = combo & 255
            g1 = plsc.load_gather(row_v, [c1])
            g2 = plsc.load_gather(row_v, [c2])
            r1 = (((lax.shift_right_logical(g1, 8) == iv) | ((g1 & 255) == iv))
                  & (c1 != iv))
            r2 = (((lax.shift_right_logical(g2, 8) == iv) | ((g2 & 255) == iv))
                  & (c2 != iv))
            n1_v[pl.ds(base, _L)] = jnp.where(r1, c1, iv)
            n2_v[pl.ds(base, _L)] = jnp.where(r2, c2, iv)
            return deg + r1.astype(jnp.int32) + r2.astype(jnp.int32)

        deg = lax.fori_loop(0, n_chunks, setup_chunk,
                            jnp.zeros((_L,), jnp.int32))
        deg_total = jnp.sum(deg)

        fv_v[...] = jnp.zeros((_L,), jnp.float32)

        @pl.when(deg_total >= 2 * (_N - 1))
        def _component_search():
            _run_component_search(deg_total, lanes, n_chunks,
                                  n1_v, n2_v, mval_v, nxt_v, fv_v)

        fv = fv_v[...][0]
        d = fv - _OPTI
        partial = partial + bce_val + jnp.float32(_COEFF) * d * d

    res_v[...] = jnp.where(lanes == 0, partial, jnp.float32(0.0))
    pltpu.sync_copy(res_v, part_sh.at[pl.ds(sid * _L, _L)])
    plsc.subcore_barrier()

    @pl.when(sid == 0)
    def _():
        pltpu.sync_copy(part_sh, all_v)
        acc = jnp.zeros((_L,), jnp.float32)
        for j in range(_NS):
            acc = acc + all_v[pl.ds(j * _L, _L)]
        res_v[...] = acc * jnp.float32(1.0 / _BS)
        pltpu.sync_copy(res_v, out_hbm.at[cid])


def _run_component_search(deg_total, lanes, n_chunks,
                          n1_v, n2_v, mval_v, nxt_v, fv_v):
        def edge_chunk(i, _):
            base = i * _L
            iv = base + lanes
            n1 = n1_v[pl.ds(base, _L)]
            n2 = n2_v[pl.ds(base, _L)]
            n1_of_n1 = plsc.load_gather(n1_v, [n1])
            n1_of_n2 = plsc.load_gather(n1_v, [n2])
            nxt1 = jnp.where(n1 == iv, iv,
                             jnp.where(n1_of_n1 == iv, n1 + _N, n1))
            nxt2 = jnp.where(n2 == iv, iv + _N,
                             jnp.where(n1_of_n2 == iv, n2 + _N, n2))
            mval_v[pl.ds(base, _L)] = n1
            mval_v[pl.ds(base + _N, _L)] = n2
            nxt_v[pl.ds(base, _L)] = nxt1
            nxt_v[pl.ds(base + _N, _L)] = nxt2
            return 0

        lax.fori_loop(0, n_chunks, edge_chunk, 0)

        def double_round(t, _):
            def dbl_chunk(i, _):
                base = i * _L
                cm = mval_v[pl.ds(base, _L)]
                nx = nxt_v[pl.ds(base, _L)]
                gm = plsc.load_gather(mval_v, [nx])
                gn = plsc.load_gather(nxt_v, [nx])
                mval_v[pl.ds(base, _L)] = jnp.minimum(cm, gm)
                nxt_v[pl.ds(base, _L)] = gn
                return 0
            lax.fori_loop(0, 2 * _N // _L, dbl_chunk, 0)
            return 0

        lax.fori_loop(0, _T_DOUBLE, double_round, 0)

        def final_chunk(i, mx):
            base = i * _L
            iv = base + lanes
            cm = jnp.minimum(iv, jnp.minimum(mval_v[pl.ds(base, _L)],
                                             mval_v[pl.ds(base + _N, _L)]))
            return jnp.maximum(mx, cm)

        mx = lax.fori_loop(0, n_chunks, final_chunk,
                           jnp.zeros((_L,), jnp.int32))
        connected = jnp.max(mx) == 0
        fv = jnp.where(connected,
                       jnp.where(deg_total == 2 * _N, _LAM_CYCLE, _LAM_PATH),
                       jnp.float32(0.0))
        fv_v[...] = jnp.where(lanes == 0, fv, jnp.float32(0.0))


@functools.cache
def _get_sc_call():
    return pl.kernel(
        _sc_body,
        out_type=jax.ShapeDtypeStruct((_NC, _L), jnp.float32),
        mesh=plsc.VectorSubcoreMesh(core_axis_name="c", subcore_axis_name="s",
                                    num_cores=_NC, num_subcores=_NS),
        scratch_types=[
            pltpu.VMEM((_PACK,), jnp.int32),
            pltpu.VMEM((_N,), jnp.int32),
            pltpu.VMEM((_N,), jnp.int32),
            pltpu.VMEM((2 * _N,), jnp.int32),
            pltpu.VMEM((2 * _N,), jnp.int32),
            pltpu.VMEM((_L,), jnp.float32),
            pltpu.VMEM((_NS * _L,), jnp.float32),
            pltpu.VMEM((_L,), jnp.float32),
            pltpu.VMEM_SHARED((_NS * _L,), jnp.float32),
        ],
        compiler_params=pltpu.CompilerParams(needs_layout_passes=False),
    )


@jax.jit
def kernel(raw_scores, target):
    packed = _tc_call(raw_scores, target)
    out = _get_sc_call()(packed.reshape(_BS, _PACK))
    return out[0, 0] + out[1, 0]


# speedup vs baseline: 1.2562x; 1.1128x over previous
import functools

import numpy as np
import jax
import jax.numpy as jnp
from jax import lax
from jax.experimental import pallas as pl
from jax.experimental.pallas import tpu as pltpu
from jax.experimental.pallas import tpu_sc as plsc

_BS = 128
_N = 256
_L = 16
_NC = 2
_NS = 16
_SAMPLES_PER_SUBCORE = _BS // (_NC * _NS)
_T_DOUBLE = 9
_PACK = _N + _L

_COEFF = 0.01
_LAM_PATH = np.float32(2.0 - 2.0 * np.cos(np.pi / _N))
_LAM_CYCLE = np.float32(2.0 - 2.0 * np.cos(2.0 * np.pi / _N))
_OPTI = _LAM_CYCLE


_B_BLK = 8


def _tc_body(x_ref, t_ref, out_ref):
    x = x_ref[...]
    t = t_ref[...]
    ax = jnp.abs(x)
    l = jnp.log(1.0 + jnp.exp(-ax))
    px = jnp.minimum(jnp.maximum(x, 0.0) + l, 100.0)
    xc = jnp.clip(x, -100.0, 100.0)
    s = (jnp.sum(px, axis=(1, 2), keepdims=True)
         - jnp.sum(t * xc, axis=(1, 2), keepdims=True)) * np.float32(1.0 / (_N * _N))

    iota = lax.broadcasted_iota(jnp.int32, (_B_BLK, _N, _N), 2)
    m1 = jnp.max(x, axis=2, keepdims=True)
    i1 = jnp.min(jnp.where(x == m1, iota, _N), axis=2, keepdims=True)
    x2 = jnp.where(iota == i1, -jnp.inf, x)
    m2 = jnp.max(x2, axis=2, keepdims=True)
    i2 = jnp.min(jnp.where(x2 == m2, iota, _N), axis=2, keepdims=True)

    combo = i1 * 256 + i2
    sbits = lax.bitcast_convert_type(s, jnp.int32)
    pad = jnp.zeros((_B_BLK, _PACK - _N - 1, 1), jnp.int32)
    out_ref[...] = jnp.concatenate([combo, sbits, pad], axis=1)[..., 0]


def _tc_call(raw_scores, target):
    return pl.pallas_call(
        _tc_body,
        grid=(_BS // _B_BLK,),
        in_specs=[
            pl.BlockSpec((_B_BLK, _N, _N), lambda b: (b, 0, 0)),
            pl.BlockSpec((_B_BLK, _N, _N), lambda b: (b, 0, 0)),
        ],
        out_specs=pl.BlockSpec((_B_BLK, _PACK), lambda b: (b, 0)),
        out_shape=jax.ShapeDtypeStruct((_BS, _PACK), jnp.int32),
        compiler_params=pltpu.CompilerParams(
            dimension_semantics=("parallel",),
        ),
    )(raw_scores, target)


def _sc_body(packed_hbm, out_hbm,
             row_v, n1_v, n2_v, mval_v, nxt_v, res_v, all_v, fv_v,
             part_sh):
    cid = lax.axis_index("c")
    sid = lax.axis_index("s")
    lanes = lax.iota(jnp.int32, _L)
    n_chunks = _N // _L

    partial = jnp.float32(0.0)
    for k in range(_SAMPLES_PER_SUBCORE):
        b = (cid * _NS + sid) * _SAMPLES_PER_SUBCORE + k
        pltpu.sync_copy(packed_hbm.at[b], row_v)
        bce_val = plsc.bitcast(row_v[pl.ds(_N, _L)], jnp.float32)[0]

        def setup_chunk(i, deg):
            base = i * _L
            iv = base + lanes
            combo = row_v[pl.ds(base, _L)]
            c1 = lax.shift_right_logical(combo, 8)
            c2 = combo & 255
            g1 = plsc.load_gather(row_v, [c1])
            g2 = plsc.load_gather(row_v, [c2])
            r1 = (((lax.shift_right_logical(g1, 8) == iv) | ((g1 & 255) == iv))
                  & (c1 != iv))
            r2 = (((lax.shift_right_logical(g2, 8) == iv) | ((g2 & 255) == iv))
                  & (c2 != iv))
            n1_v[pl.ds(base, _L)] = jnp.where(r1, c1, iv)
            n2_v[pl.ds(base, _L)] = jnp.where(r2, c2, iv)
            return deg + r1.astype(jnp.int32) + r2.astype(jnp.int32)

        deg = lax.fori_loop(0, n_chunks, setup_chunk,
                            jnp.zeros((_L,), jnp.int32))
        deg_total = jnp.sum(deg)

        fv_v[...] = jnp.zeros((_L,), jnp.float32)

        @pl.when(deg_total >= 2 * (_N - 1))
        def _component_search():
            _run_component_search(deg_total, lanes, n_chunks,
                                  n1_v, n2_v, mval_v, nxt_v, fv_v)

        fv = fv_v[...][0]
        d = fv - _OPTI
        partial = partial + bce_val + jnp.float32(_COEFF) * d * d

    res_v[...] = jnp.where(lanes == 0, partial, jnp.float32(0.0))
    pltpu.sync_copy(res_v, part_sh.at[pl.ds(sid * _L, _L)])
    plsc.subcore_barrier()

    @pl.when(sid == 0)
    def _():
        pltpu.sync_copy(part_sh, all_v)
        acc = jnp.zeros((_L,), jnp.float32)
        for j in range(_NS):
            acc = acc + all_v[pl.ds(j * _L, _L)]
        res_v[...] = acc * jnp.float32(1.0 / _BS)
        pltpu.sync_copy(res_v, out_hbm.at[cid])


def _run_component_search(deg_total, lanes, n_chunks,
                          n1_v, n2_v, mval_v, nxt_v, fv_v):
        def edge_chunk(i, _):
            base = i * _L
            iv = base + lanes
            n1 = n1_v[pl.ds(base, _L)]
            n2 = n2_v[pl.ds(base, _L)]
            n1_of_n1 = plsc.load_gather(n1_v, [n1])
            n1_of_n2 = plsc.load_gather(n1_v, [n2])
            nxt1 = jnp.where(n1 == iv, iv,
                             jnp.where(n1_of_n1 == iv, n1 + _N, n1))
            nxt2 = jnp.where(n2 == iv, iv + _N,
                             jnp.where(n1_of_n2 == iv, n2 + _N, n2))
            mval_v[pl.ds(base, _L)] = n1
            mval_v[pl.ds(base + _N, _L)] = n2
            nxt_v[pl.ds(base, _L)] = nxt1
            nxt_v[pl.ds(base + _N, _L)] = nxt2
            return 0

        lax.fori_loop(0, n_chunks, edge_chunk, 0)

        def double_round(t, _):
            def dbl_chunk(i, _):
                base = i * _L
                cm = mval_v[pl.ds(base, _L)]
                nx = nxt_v[pl.ds(base, _L)]
                gm = plsc.load_gather(mval_v, [nx])
                gn = plsc.load_gather(nxt_v, [nx])
                mval_v[pl.ds(base, _L)] = jnp.minimum(cm, gm)
                nxt_v[pl.ds(base, _L)] = gn
                return 0
            lax.fori_loop(0, 2 * _N // _L, dbl_chunk, 0)
            return 0

        lax.fori_loop(0, _T_DOUBLE, double_round, 0)

        def final_chunk(i, mx):
            base = i * _L
            iv = base + lanes
            cm = jnp.minimum(iv, jnp.minimum(mval_v[pl.ds(base, _L)],
                                             mval_v[pl.ds(base + _N, _L)]))
            return jnp.maximum(mx, cm)

        mx = lax.fori_loop(0, n_chunks, final_chunk,
                           jnp.zeros((_L,), jnp.int32))
        connected = jnp.max(mx) == 0
        fv = jnp.where(connected,
                       jnp.where(deg_total == 2 * _N, _LAM_CYCLE, _LAM_PATH),
                       jnp.float32(0.0))
        fv_v[...] = jnp.where(lanes == 0, fv, jnp.float32(0.0))


@functools.cache
def _get_sc_call():
    return pl.kernel(
        _sc_body,
        out_type=jax.ShapeDtypeStruct((_NC, _L), jnp.float32),
        mesh=plsc.VectorSubcoreMesh(core_axis_name="c", subcore_axis_name="s",
                                    num_cores=_NC, num_subcores=_NS),
        scratch_types=[
            pltpu.VMEM((_PACK,), jnp.int32),
            pltpu.VMEM((_N,), jnp.int32),
            pltpu.VMEM((_N,), jnp.int32),
            pltpu.VMEM((2 * _N,), jnp.int32),
            pltpu.VMEM((2 * _N,), jnp.int32),
            pltpu.VMEM((_L,), jnp.float32),
            pltpu.VMEM((_NS * _L,), jnp.float32),
            pltpu.VMEM((_L,), jnp.float32),
            pltpu.VMEM_SHARED((_NS * _L,), jnp.float32),
        ],
        compiler_params=pltpu.CompilerParams(needs_layout_passes=False),
    )


@jax.jit
def kernel(raw_scores, target):
    packed = _tc_call(raw_scores, target)
    out = _get_sc_call()(packed)
    return out[0, 0] + out[1, 0]
